# Initial kernel scaffold; baseline (speedup 1.0000x reference)
import jax
import jax.numpy as jnp
from jax.experimental import pallas as pl


def kernel(x_nchw, conv1_w, conv1_b, down1_w, down1_b, down2_w, down2_b, down3_w, down3_b, conv2_w, conv2_b):
    raise NotImplementedError("write your pallas kernel here")


# speedup vs baseline: 1.4882x; 1.4882x over previous
import functools

import jax
import jax.numpy as jnp
from jax.experimental import pallas as pl
from jax.experimental.pallas import tpu as pltpu


def _down_on_value(v, wk, bb, Hc, Wc, C):
    Ho, Wo = Hc // 2, Wc // 2
    v5 = v.reshape(Ho, 2, Wo, 2, C)
    taps = [v5[:, ki, :, kj, :] for ki in (0, 1) for kj in (0, 1)]
    pooled = jnp.maximum(jnp.maximum(taps[0], taps[1]),
                         jnp.maximum(taps[2], taps[3]))
    dn = bb
    for t, (ki, kj) in zip(taps, ((0, 0), (0, 1), (1, 0), (1, 1))):
        dn = dn + jnp.dot(t.reshape(Ho * Wo, C), wk[ki, kj],
                          preferred_element_type=jnp.float32)
    return pooled.reshape(Ho, Wo, C) + dn.reshape(Ho, Wo, C)


def _fused_body(x0_ref, x1_ref, x2_ref, w1_ref, b1_ref, wk1_ref, bb1_ref,
                wk2_ref, bb2_ref, wk3_ref, bb3_ref, w2_ref, b2_ref, o_ref,
                d3_ref, *, H, W, strip, Cin_p):
    n_strips = H // strip
    s = pl.program_id(1)

    xs = jnp.concatenate([x0_ref[0], x1_ref[0], x2_ref[0]], axis=0)
    taps = []
    for di in range(3):
        for dj in range(3):
            taps.append(xs[di:di + strip, dj:dj + W, :].reshape(strip * W,
                                                               Cin_p))
    a = jnp.concatenate(taps, axis=1)
    c1 = jnp.dot(a, w1_ref[...],
                 preferred_element_type=jnp.float32) + b1_ref[...]

    d1 = _down_on_value(c1.reshape(strip, W, 64), wk1_ref[...], bb1_ref[...],
                        strip, W, 64)
    d2 = _down_on_value(d1, wk2_ref[...], bb2_ref[...],
                        strip // 2, W // 2, 64)
    d3_ref[pl.ds(s * (strip // 8), strip // 8)] = _down_on_value(
        d2, wk3_ref[...], bb3_ref[...], strip // 4, W // 4, 64)

    @pl.when(s == n_strips - 1)
    def _tail():
        d3 = d3_ref[...]
        H8, W8 = H // 8, W // 8
        d3p = jnp.pad(d3, ((1, 1), (1, 1), (0, 0)))
        taps2 = [d3p[di:di + H8, dj:dj + W8, :].reshape(H8 * W8, 64)
                 for di in range(3) for dj in range(3)]
        a2 = jnp.concatenate(taps2, axis=1)
        out = jnp.dot(a2, w2_ref[...],
                      preferred_element_type=jnp.float32) + b2_ref[...]
        o_ref[0] = out.reshape(H8, W8, 8)


@jax.jit
def kernel(x_nchw, conv1_w, conv1_b, down1_w, down1_b, down2_w, down2_b,
           down3_w, down3_b, conv2_w, conv2_b):
    N, Cin, H, W = x_nchw.shape
    Cin_p = 8
    C = down1_w.shape[-1]
    H8, W8 = H // 8, W // 8
    Cout = conv2_w.shape[-1]
    strip = 32
    n_strips = H // strip
    hb = strip // 2

    x = jnp.transpose(x_nchw, (0, 2, 3, 1)).astype(jnp.float32)
    x = jnp.pad(x, ((0, 0), (1, 2 * hb - 1), (1, 1), (0, Cin_p - Cin)))
    w1 = jnp.pad(conv1_w.astype(jnp.float32),
                 ((0, 0), (0, 0), (0, Cin_p - Cin), (0, 0)))
    w1 = w1.reshape(9 * Cin_p, C)
    b1 = conv1_b.reshape(1, C).astype(jnp.float32)
    wk1 = down1_w.astype(jnp.float32)
    wk2 = down2_w.astype(jnp.float32)
    wk3 = down3_w.astype(jnp.float32)
    bb1 = down1_b.reshape(1, C).astype(jnp.float32)
    bb2 = down2_b.reshape(1, C).astype(jnp.float32)
    bb3 = down3_b.reshape(1, C).astype(jnp.float32)
    w2 = jnp.pad(conv2_w.reshape(9 * C, Cout).astype(jnp.float32),
                 ((0, 0), (0, 8 - Cout)))
    b2 = jnp.pad(conv2_b.astype(jnp.float32), ((0, 8 - Cout),)).reshape(1, 8)

    body = functools.partial(_fused_body, H=H, W=W, strip=strip, Cin_p=Cin_p)
    zero2 = lambda i, s: (0, 0)
    zero4 = lambda i, s: (0, 0, 0, 0)
    xspec = lambda k: pl.BlockSpec((1, hb, W + 2, Cin_p),
                                   lambda i, s, k=k: (i, 2 * s + k, 0, 0))
    out = pl.pallas_call(
        body,
        out_shape=jax.ShapeDtypeStruct((N, H8, W8, 8), jnp.float32),
        grid=(N, n_strips),
        in_specs=[
            xspec(0), xspec(1), xspec(2),
            pl.BlockSpec((9 * Cin_p, C), zero2),
            pl.BlockSpec((1, C), zero2),
            pl.BlockSpec((2, 2, C, C), zero4),
            pl.BlockSpec((1, C), zero2),
            pl.BlockSpec((2, 2, C, C), zero4),
            pl.BlockSpec((1, C), zero2),
            pl.BlockSpec((2, 2, C, C), zero4),
            pl.BlockSpec((1, C), zero2),
            pl.BlockSpec((9 * C, 8), zero2),
            pl.BlockSpec((1, 8), zero2),
        ],
        out_specs=pl.BlockSpec((1, H8, W8, 8), lambda i, s: (i, 0, 0, 0)),
        scratch_shapes=[pltpu.VMEM((H // 8, W // 8, C), jnp.float32)],
        compiler_params=pltpu.CompilerParams(
            dimension_semantics=("parallel", "arbitrary"),
            vmem_limit_bytes=64 * 1024 * 1024,
        ),
    )(x, x, x, w1, b1, wk1, bb1, wk2, bb2, wk3, bb3, w2, b2)

    return jnp.transpose(out[..., :Cout], (0, 3, 1, 2))


# speedup vs baseline: 2.7853x; 1.8717x over previous
import functools

import jax
import jax.numpy as jnp
from jax.experimental import pallas as pl
from jax.experimental.pallas import tpu as pltpu


def _down_phase_group(phases, n_out_phase, wk, bb, C):
    rows = phases[0].shape[0]
    cols = phases[0].shape[2]
    m = rows * cols
    blocks = []
    pooled = []
    for r in range(n_out_phase):
        taps = [phases[2 * r + kj][:, ki] for ki in (0, 1) for kj in (0, 1)]
        pooled.append(jnp.maximum(jnp.maximum(taps[0], taps[1]),
                                  jnp.maximum(taps[2], taps[3])))
        blocks.append(jnp.concatenate(
            [t.reshape(m, C) for t in taps], axis=1))
    a = jnp.concatenate(blocks, axis=0)
    dn = jnp.dot(a, wk, preferred_element_type=jnp.float32) + bb
    pool = jnp.concatenate([p.reshape(m, C) for p in pooled], axis=0)
    return (dn + pool).reshape(n_out_phase, rows, cols, C)


def _fused_body(x0_ref, x1_ref, x2_ref, w1_ref, b1_ref, wk1_ref, bb1_ref,
                wk2_ref, bb2_ref, wk3_ref, bb3_ref, w2_ref, b2_ref, o_ref,
                d3_ref, *, H, W, strip, Cin_p, pbw):
    n_strips = H // strip
    wc = W // 8
    s = pl.program_id(1)

    xs = jnp.concatenate([x0_ref[0], x1_ref[0], x2_ref[0]], axis=0)
    ap = []
    for p in range(8):
        taps = []
        for di in range(3):
            for dj in range(3):
                q, o = (p + dj) % 8, (p + dj) // 8
                taps.append(
                    xs[di:di + strip, pbw * q + o:pbw * q + o + wc, :]
                    .reshape(strip * wc, Cin_p))
        ap.append(jnp.concatenate(taps, axis=1))
    a1 = jnp.concatenate(ap, axis=0)
    c1 = jnp.dot(a1, w1_ref[...],
                 preferred_element_type=jnp.float32) + b1_ref[...]
    c1 = c1.reshape(8, strip // 2, 2, wc, 64)

    d1 = _down_phase_group([c1[p] for p in range(8)], 4,
                           wk1_ref[...], bb1_ref[...], 64)
    d1 = d1.reshape(4, strip // 4, 2, wc, 64)
    d2 = _down_phase_group([d1[p] for p in range(4)], 2,
                           wk2_ref[...], bb2_ref[...], 64)
    d2 = d2.reshape(2, strip // 8, 2, wc, 64)
    d3 = _down_phase_group([d2[p] for p in range(2)], 1,
                           wk3_ref[...], bb3_ref[...], 64)
    d3_ref[pl.ds(s * (strip // 8), strip // 8)] = d3.reshape(
        strip // 8, wc, 64)

    @pl.when(s == n_strips - 1)
    def _tail():
        H8, W8 = H // 8, W // 8
        d3p = jnp.pad(d3_ref[...], ((1, 1), (1, 1), (0, 0)))
        taps2 = [d3p[di:di + H8, dj:dj + W8, :].reshape(H8 * W8, 64)
                 for di in range(3) for dj in range(3)]
        a2 = jnp.concatenate(taps2, axis=1)
        out = jnp.dot(a2, w2_ref[...],
                      preferred_element_type=jnp.float32) + b2_ref[...]
        o_ref[0] = out.reshape(H8, W8, 8)


@jax.jit
def kernel(x_nchw, conv1_w, conv1_b, down1_w, down1_b, down2_w, down2_b,
           down3_w, down3_b, conv2_w, conv2_b):
    N, Cin, H, W = x_nchw.shape
    Cin_p = 8
    C = down1_w.shape[-1]
    H8, W8 = H // 8, W // 8
    Cout = conv2_w.shape[-1]
    strip = 32
    n_strips = H // strip
    hb = strip // 2
    pbw = W // 8 + 1

    x = jnp.transpose(x_nchw, (0, 2, 3, 1)).astype(jnp.float32)
    x = jnp.pad(x, ((0, 0), (1, 2 * hb - 1), (1, 8 * pbw - W - 1),
                    (0, Cin_p - Cin)))
    R = x.shape[1]
    x = x.reshape(N, R, pbw, 8, Cin_p).transpose(0, 1, 3, 2, 4)
    x = x.reshape(N, R, 8 * pbw, Cin_p)
    w1 = jnp.pad(conv1_w.astype(jnp.float32),
                 ((0, 0), (0, 0), (0, Cin_p - Cin), (0, 0)))
    w1 = w1.reshape(9 * Cin_p, C)
    b1 = conv1_b.reshape(1, C).astype(jnp.float32)
    wk1 = down1_w.astype(jnp.float32).reshape(4 * C, C)
    wk2 = down2_w.astype(jnp.float32).reshape(4 * C, C)
    wk3 = down3_w.astype(jnp.float32).reshape(4 * C, C)
    bb1 = down1_b.reshape(1, C).astype(jnp.float32)
    bb2 = down2_b.reshape(1, C).astype(jnp.float32)
    bb3 = down3_b.reshape(1, C).astype(jnp.float32)
    w2 = jnp.pad(conv2_w.reshape(9 * C, Cout).astype(jnp.float32),
                 ((0, 0), (0, 8 - Cout)))
    b2 = jnp.pad(conv2_b.astype(jnp.float32), ((0, 8 - Cout),)).reshape(1, 8)

    body = functools.partial(_fused_body, H=H, W=W, strip=strip,
                             Cin_p=Cin_p, pbw=pbw)
    zero2 = lambda i, s: (0, 0)
    xspec = lambda k: pl.BlockSpec((1, hb, 8 * pbw, Cin_p),
                                   lambda i, s, k=k: (i, 2 * s + k, 0, 0))
    out = pl.pallas_call(
        body,
        out_shape=jax.ShapeDtypeStruct((N, H8, W8, 8), jnp.float32),
        grid=(N, n_strips),
        in_specs=[
            xspec(0), xspec(1), xspec(2),
            pl.BlockSpec((9 * Cin_p, C), zero2),
            pl.BlockSpec((1, C), zero2),
            pl.BlockSpec((4 * C, C), zero2),
            pl.BlockSpec((1, C), zero2),
            pl.BlockSpec((4 * C, C), zero2),
            pl.BlockSpec((1, C), zero2),
            pl.BlockSpec((4 * C, C), zero2),
            pl.BlockSpec((1, C), zero2),
            pl.BlockSpec((9 * C, 8), zero2),
            pl.BlockSpec((1, 8), zero2),
        ],
        out_specs=pl.BlockSpec((1, H8, W8, 8), lambda i, s: (i, 0, 0, 0)),
        scratch_shapes=[pltpu.VMEM((H8, W8, C), jnp.float32)],
        compiler_params=pltpu.CompilerParams(
            dimension_semantics=("parallel", "arbitrary"),
            vmem_limit_bytes=64 * 1024 * 1024,
        ),
    )(x, x, x, w1, b1, wk1, bb1, wk2, bb2, wk3, bb3, w2, b2)

    return jnp.transpose(out[..., :Cout], (0, 3, 1, 2))


# speedup vs baseline: 4.3462x; 1.5604x over previous
import functools

import jax
import jax.numpy as jnp
from jax.experimental import pallas as pl
from jax.experimental.pallas import tpu as pltpu


def _down_phase_group(phases, n_out_phase, wk, bb, C):
    rows = phases[0].shape[0]
    cols = phases[0].shape[2]
    m = rows * cols
    blocks = []
    pooled = []
    for r in range(n_out_phase):
        taps = [phases[2 * r + kj][:, ki] for ki in (0, 1) for kj in (0, 1)]
        pooled.append(jnp.maximum(jnp.maximum(taps[0], taps[1]),
                                  jnp.maximum(taps[2], taps[3])))
        blocks.append(jnp.concatenate(
            [t.reshape(m, C) for t in taps], axis=1))
    a = jnp.concatenate(blocks, axis=0)
    dn = jnp.dot(a, wk, preferred_element_type=jnp.float32) + bb
    pool = jnp.concatenate([p.reshape(m, C) for p in pooled], axis=0)
    return (dn + pool).reshape(n_out_phase, rows, cols, C)


def _fused_body(x0_ref, x1_ref, x2_ref, w1_ref, b1_ref, wk1_ref, bb1_ref,
                wk2_ref, bb2_ref, wk3_ref, bb3_ref, w2_ref, b2_ref, o_ref,
                d3_ref, *, H, W, strip, Cin_p, pbw):
    n_strips = H // strip
    wc = W // 8
    s = pl.program_id(1)

    xs = jnp.concatenate([x0_ref[0], x1_ref[0], x2_ref[0]], axis=0)
    ap = []
    for p in range(8):
        taps = [
            xs[di:di + strip, wc * p:wc * p + wc, :]
            .reshape(strip * wc, 3 * Cin_p)
            for di in range(3)
        ]
        ap.append(jnp.concatenate(taps, axis=1))
    a1 = jnp.concatenate(ap, axis=0)
    c1 = jnp.dot(a1, w1_ref[...],
                 preferred_element_type=jnp.float32) + b1_ref[...]
    c1 = c1.reshape(8, strip // 2, 2, wc, 64)

    d1 = _down_phase_group([c1[p] for p in range(8)], 4,
                           wk1_ref[...], bb1_ref[...], 64)
    d1 = d1.reshape(4, strip // 4, 2, wc, 64)
    d2 = _down_phase_group([d1[p] for p in range(4)], 2,
                           wk2_ref[...], bb2_ref[...], 64)
    d2 = d2.reshape(2, strip // 8, 2, wc, 64)
    d3 = _down_phase_group([d2[p] for p in range(2)], 1,
                           wk3_ref[...], bb3_ref[...], 64)
    d3_ref[pl.ds(s * (strip // 8), strip // 8)] = d3.reshape(
        strip // 8, wc, 64)

    @pl.when(s == n_strips - 1)
    def _tail():
        H8, W8 = H // 8, W // 8
        d3p = jnp.pad(d3_ref[...], ((1, 1), (1, 1), (0, 0)))
        taps2 = [d3p[di:di + H8, dj:dj + W8, :].reshape(H8 * W8, 64)
                 for di in range(3) for dj in range(3)]
        a2 = jnp.concatenate(taps2, axis=1)
        out = jnp.dot(a2, w2_ref[...],
                      preferred_element_type=jnp.float32) + b2_ref[...]
        o_ref[0] = out.reshape(H8, W8, 8)


@jax.jit
def kernel(x_nchw, conv1_w, conv1_b, down1_w, down1_b, down2_w, down2_b,
           down3_w, down3_b, conv2_w, conv2_b):
    N, Cin, H, W = x_nchw.shape
    Cin_p = 8
    C = down1_w.shape[-1]
    H8, W8 = H // 8, W // 8
    Cout = conv2_w.shape[-1]
    strip = 32
    n_strips = H // strip
    hb = strip // 2
    pbw = W // 8

    x = jnp.transpose(x_nchw, (0, 2, 3, 1)).astype(jnp.float32)
    x = jnp.pad(x, ((0, 0), (1, 2 * hb - 1), (1, 1), (0, Cin_p - Cin)))
    R = x.shape[1]
    xd = [x[:, :, dj:dj + W, :].reshape(N, R, pbw, 8, Cin_p)
          .transpose(0, 1, 3, 2, 4) for dj in range(3)]
    x = jnp.stack(xd, axis=-2)
    x = x.reshape(N, R, 8 * pbw, 3 * Cin_p)
    w1 = jnp.pad(conv1_w.astype(jnp.float32),
                 ((0, 0), (0, 0), (0, Cin_p - Cin), (0, 0)))
    w1 = w1.reshape(9 * Cin_p, C)
    b1 = conv1_b.reshape(1, C).astype(jnp.float32)
    wk1 = down1_w.astype(jnp.float32).reshape(4 * C, C)
    wk2 = down2_w.astype(jnp.float32).reshape(4 * C, C)
    wk3 = down3_w.astype(jnp.float32).reshape(4 * C, C)
    bb1 = down1_b.reshape(1, C).astype(jnp.float32)
    bb2 = down2_b.reshape(1, C).astype(jnp.float32)
    bb3 = down3_b.reshape(1, C).astype(jnp.float32)
    w2 = jnp.pad(conv2_w.reshape(9 * C, Cout).astype(jnp.float32),
                 ((0, 0), (0, 8 - Cout)))
    b2 = jnp.pad(conv2_b.astype(jnp.float32), ((0, 8 - Cout),)).reshape(1, 8)

    body = functools.partial(_fused_body, H=H, W=W, strip=strip,
                             Cin_p=Cin_p, pbw=pbw)
    zero2 = lambda i, s: (0, 0)
    xspec = lambda k: pl.BlockSpec((1, hb, 8 * pbw, 3 * Cin_p),
                                   lambda i, s, k=k: (i, 2 * s + k, 0, 0))
    out = pl.pallas_call(
        body,
        out_shape=jax.ShapeDtypeStruct((N, H8, W8, 8), jnp.float32),
        grid=(N, n_strips),
        in_specs=[
            xspec(0), xspec(1), xspec(2),
            pl.BlockSpec((9 * Cin_p, C), zero2),
            pl.BlockSpec((1, C), zero2),
            pl.BlockSpec((4 * C, C), zero2),
            pl.BlockSpec((1, C), zero2),
            pl.BlockSpec((4 * C, C), zero2),
            pl.BlockSpec((1, C), zero2),
            pl.BlockSpec((4 * C, C), zero2),
            pl.BlockSpec((1, C), zero2),
            pl.BlockSpec((9 * C, 8), zero2),
            pl.BlockSpec((1, 8), zero2),
        ],
        out_specs=pl.BlockSpec((1, H8, W8, 8), lambda i, s: (i, 0, 0, 0)),
        scratch_shapes=[pltpu.VMEM((H8, W8, C), jnp.float32)],
        compiler_params=pltpu.CompilerParams(
            dimension_semantics=("parallel", "arbitrary"),
            vmem_limit_bytes=64 * 1024 * 1024,
        ),
    )(x, x, x, w1, b1, wk1, bb1, wk2, bb2, wk3, bb3, w2, b2)

    return jnp.transpose(out[..., :Cout], (0, 3, 1, 2))


# speedup vs baseline: 6.3382x; 1.4583x over previous
import functools

import jax
import jax.numpy as jnp
from jax.experimental import pallas as pl
from jax.experimental.pallas import tpu as pltpu


def _down_phase_group(phases, n_out_phase, wk, bb, C):
    rows = phases[0].shape[0]
    cols = phases[0].shape[2]
    m = rows * cols
    blocks = []
    pooled = []
    for r in range(n_out_phase):
        taps = [phases[2 * r + kj][:, ki] for ki in (0, 1) for kj in (0, 1)]
        pooled.append(jnp.maximum(jnp.maximum(taps[0], taps[1]),
                                  jnp.maximum(taps[2], taps[3])))
        blocks.append(jnp.concatenate(
            [t.reshape(m, C) for t in taps], axis=1))
    a = jnp.concatenate(blocks, axis=0)
    dn = jnp.dot(a, wk, preferred_element_type=jnp.float32) + bb
    pool = jnp.concatenate([p.reshape(m, C) for p in pooled], axis=0)
    out = dn + pool.astype(jnp.float32)
    return out.astype(jnp.bfloat16).reshape(n_out_phase, rows, cols, C)


def _fused_body(x0_ref, x1_ref, x2_ref, w1_ref, b1_ref, wk1_ref, bb1_ref,
                wk2_ref, bb2_ref, wk3_ref, bb3_ref, w2_ref, b2_ref, o_ref,
                d3_ref, *, H, W, strip, Cin_p, pbw):
    n_strips = H // strip
    wc = W // 8
    s = pl.program_id(1)

    xs = jnp.concatenate([x0_ref[0], x1_ref[0], x2_ref[0]], axis=0)
    ap = []
    for p in range(8):
        taps = [
            xs[di:di + strip, wc * p:wc * p + wc, :]
            .reshape(strip * wc, 3 * Cin_p)
            for di in range(3)
        ]
        ap.append(jnp.concatenate(taps, axis=1))
    a1 = jnp.concatenate(ap, axis=0)
    c1 = jnp.dot(a1, w1_ref[...],
                 preferred_element_type=jnp.float32) + b1_ref[...]
    c1 = c1.astype(jnp.bfloat16)
    c1 = c1.reshape(8, strip // 2, 2, wc, 64)

    d1 = _down_phase_group([c1[p] for p in range(8)], 4,
                           wk1_ref[...], bb1_ref[...], 64)
    d1 = d1.reshape(4, strip // 4, 2, wc, 64)
    d2 = _down_phase_group([d1[p] for p in range(4)], 2,
                           wk2_ref[...], bb2_ref[...], 64)
    d2 = d2.reshape(2, strip // 8, 2, wc, 64)
    d3 = _down_phase_group([d2[p] for p in range(2)], 1,
                           wk3_ref[...], bb3_ref[...], 64)
    d3_ref[pl.ds(s * (strip // 8), strip // 8)] = d3.reshape(
        strip // 8, wc, 64)

    @pl.when(s == n_strips - 1)
    def _tail():
        H8, W8 = H // 8, W // 8
        d3p = jnp.pad(d3_ref[...], ((1, 1), (1, 1), (0, 0)))
        taps2 = [d3p[di:di + H8, dj:dj + W8, :].reshape(H8 * W8, 64)
                 for di in range(3) for dj in range(3)]
        a2 = jnp.concatenate(taps2, axis=1)
        out = jnp.dot(a2, w2_ref[...],
                      preferred_element_type=jnp.float32) + b2_ref[...]
        o_ref[0] = out.reshape(H8, W8, 8)


@jax.jit
def kernel(x_nchw, conv1_w, conv1_b, down1_w, down1_b, down2_w, down2_b,
           down3_w, down3_b, conv2_w, conv2_b):
    N, Cin, H, W = x_nchw.shape
    Cin_p = 8
    C = down1_w.shape[-1]
    H8, W8 = H // 8, W // 8
    Cout = conv2_w.shape[-1]
    strip = 32
    n_strips = H // strip
    hb = strip // 2
    pbw = W // 8

    x = jnp.transpose(x_nchw, (0, 2, 3, 1)).astype(jnp.float32)
    x = jnp.pad(x, ((0, 0), (1, 2 * hb - 1), (1, 1), (0, Cin_p - Cin)))
    R = x.shape[1]
    xd = [x[:, :, dj:dj + W, :].reshape(N, R, pbw, 8, Cin_p)
          .transpose(0, 1, 3, 2, 4) for dj in range(3)]
    x = jnp.stack(xd, axis=-2)
    x = x.reshape(N, R, 8 * pbw, 3 * Cin_p).astype(jnp.bfloat16)
    w1 = jnp.pad(conv1_w.astype(jnp.float32),
                 ((0, 0), (0, 0), (0, Cin_p - Cin), (0, 0)))
    w1 = w1.reshape(9 * Cin_p, C).astype(jnp.bfloat16)
    b1 = conv1_b.reshape(1, C).astype(jnp.float32)
    wk1 = down1_w.astype(jnp.bfloat16).reshape(4 * C, C)
    wk2 = down2_w.astype(jnp.bfloat16).reshape(4 * C, C)
    wk3 = down3_w.astype(jnp.bfloat16).reshape(4 * C, C)
    bb1 = down1_b.reshape(1, C).astype(jnp.float32)
    bb2 = down2_b.reshape(1, C).astype(jnp.float32)
    bb3 = down3_b.reshape(1, C).astype(jnp.float32)
    w2 = jnp.pad(conv2_w.reshape(9 * C, Cout).astype(jnp.bfloat16),
                 ((0, 0), (0, 8 - Cout)))
    b2 = jnp.pad(conv2_b.astype(jnp.float32), ((0, 8 - Cout),)).reshape(1, 8)

    body = functools.partial(_fused_body, H=H, W=W, strip=strip,
                             Cin_p=Cin_p, pbw=pbw)
    zero2 = lambda i, s: (0, 0)
    xspec = lambda k: pl.BlockSpec((1, hb, 8 * pbw, 3 * Cin_p),
                                   lambda i, s, k=k: (i, 2 * s + k, 0, 0))
    out = pl.pallas_call(
        body,
        out_shape=jax.ShapeDtypeStruct((N, H8, W8, 8), jnp.float32),
        grid=(N, n_strips),
        in_specs=[
            xspec(0), xspec(1), xspec(2),
            pl.BlockSpec((9 * Cin_p, C), zero2),
            pl.BlockSpec((1, C), zero2),
            pl.BlockSpec((4 * C, C), zero2),
            pl.BlockSpec((1, C), zero2),
            pl.BlockSpec((4 * C, C), zero2),
            pl.BlockSpec((1, C), zero2),
            pl.BlockSpec((4 * C, C), zero2),
            pl.BlockSpec((1, C), zero2),
            pl.BlockSpec((9 * C, 8), zero2),
            pl.BlockSpec((1, 8), zero2),
        ],
        out_specs=pl.BlockSpec((1, H8, W8, 8), lambda i, s: (i, 0, 0, 0)),
        scratch_shapes=[pltpu.VMEM((H8, W8, C), jnp.bfloat16)],
        compiler_params=pltpu.CompilerParams(
            dimension_semantics=("parallel", "arbitrary"),
            vmem_limit_bytes=64 * 1024 * 1024,
        ),
    )(x, x, x, w1, b1, wk1, bb1, wk2, bb2, wk3, bb3, w2, b2)

    return jnp.transpose(out[..., :Cout], (0, 3, 1, 2))
